# baseline (speedup 1.0000x reference)
import functools

import jax
import jax.numpy as jnp
from jax.experimental import pallas as pl
from jax.experimental.pallas import tpu as pltpu


def _manual_body(rgb_hbm, hha_hbm, w1t_ref, b1_ref, w2_ref, b2_ref,
                 out_hbm, rgb_v, hha_v, out_v, in_sems, out_sems,
                 *, spc, ck, inv_hw):
    base = pl.program_id(0) * spc
    ng = spc // ck

    for g in range(ng):
        pltpu.make_async_copy(rgb_hbm.at[pl.ds(base + g * ck, ck)],
                              rgb_v.at[pl.ds(g * ck, ck)],
                              in_sems.at[0, g]).start()
        pltpu.make_async_copy(hha_hbm.at[pl.ds(base + g * ck, ck)],
                              hha_v.at[pl.ds(g * ck, ck)],
                              in_sems.at[1, g]).start()

    srs, shs = [], []
    for g in range(ng):
        lo = g * ck
        pltpu.make_async_copy(rgb_hbm.at[pl.ds(base + lo, ck)],
                              rgb_v.at[pl.ds(lo, ck)],
                              in_sems.at[0, g]).wait()
        pltpu.make_async_copy(hha_hbm.at[pl.ds(base + lo, ck)],
                              hha_v.at[pl.ds(lo, ck)],
                              in_sems.at[1, g]).wait()
        srs.append(jnp.sum(jnp.sum(rgb_v[lo:lo + ck], axis=-1,
                                   dtype=jnp.float32), axis=-1))
        shs.append(jnp.sum(jnp.sum(hha_v[lo:lo + ck], axis=-1,
                                   dtype=jnp.float32), axis=-1))

    s = jnp.concatenate([jnp.concatenate(srs, axis=0),
                         jnp.concatenate(shs, axis=0)], axis=-1)
    h = jnp.dot(s, w1t_ref[...],
                preferred_element_type=jnp.float32)
    h = jnp.maximum(h * inv_hw + b1_ref[...][None, :], 0.0)
    w2d = w2_ref[0:1, :] - w2_ref[1:2, :]
    b2d = b2_ref[...][0:1] - b2_ref[...][1:2]
    d = jnp.sum(h * w2d, axis=-1, keepdims=True) + b2d
    w_all = jax.nn.sigmoid(d)

    for g in range(ng):
        lo = g * ck
        w_rgb = w_all[lo:lo + ck, :, None, None]
        rgb = rgb_v[lo:lo + ck]
        hha = hha_v[lo:lo + ck]
        out_v[lo:lo + ck] = (hha + w_rgb * (rgb - hha)).astype(out_v.dtype)
        pltpu.make_async_copy(out_v.at[pl.ds(lo, ck)],
                              out_hbm.at[pl.ds(base + lo, ck)],
                              out_sems.at[g]).start()

    for g in range(ng):
        lo = g * ck
        pltpu.make_async_copy(out_v.at[pl.ds(lo, ck)],
                              out_hbm.at[pl.ds(base + lo, ck)],
                              out_sems.at[g]).wait()


def kernel(rgb, hha, w1, b1, w2, b2):
    assert rgb.shape == hha.shape and rgb.dtype == hha.dtype
    B, C, H, W = rgb.shape

    w1t = w1.T

    ncores = 2 if B % 2 == 0 else 1
    spc = B // ncores
    ck = 2 if spc % 2 == 0 else 1

    def park(shape):
        return pl.BlockSpec(shape, lambda b: (0,) * len(shape))

    body = functools.partial(_manual_body, spc=spc, ck=ck,
                             inv_hw=1.0 / (H * W))
    return pl.pallas_call(
        body,
        out_shape=jax.ShapeDtypeStruct((B, C, H, W), rgb.dtype),
        grid=(ncores,),
        in_specs=[
            pl.BlockSpec(memory_space=pl.ANY),
            pl.BlockSpec(memory_space=pl.ANY),
            park(w1t.shape), park(b1.shape), park(w2.shape), park(b2.shape),
        ],
        out_specs=pl.BlockSpec(memory_space=pl.ANY),
        scratch_shapes=[
            pltpu.VMEM((spc, C, H, W), rgb.dtype),
            pltpu.VMEM((spc, C, H, W), rgb.dtype),
            pltpu.VMEM((spc, C, H, W), rgb.dtype),
            pltpu.SemaphoreType.DMA((2, spc // ck)),
            pltpu.SemaphoreType.DMA((spc // ck,)),
        ],
        compiler_params=pltpu.CompilerParams(
            dimension_semantics=("parallel",),
            vmem_limit_bytes=64 * 1024 * 1024),
        cost_estimate=pl.CostEstimate(
            flops=5 * B * C * H * W,
            transcendentals=B,
            bytes_accessed=3 * B * C * H * W * jnp.dtype(rgb.dtype).itemsize),
    )(rgb, hha, w1t, b1, w2, b2)


# speedup vs baseline: 1.1454x; 1.1454x over previous
import functools

import jax
import jax.numpy as jnp
from jax.experimental import pallas as pl
from jax.experimental.pallas import tpu as pltpu


def _fused_body(rgb_ref, hha_ref, w1_ref, b1_ref, w2_ref, b2_ref, out_ref,
                *, inv_hw):
    rgb = rgb_ref[...]
    hha = hha_ref[...]

    sr = jnp.sum(jnp.sum(rgb, axis=-1, dtype=jnp.float32), axis=-1)
    sh = jnp.sum(jnp.sum(hha, axis=-1, dtype=jnp.float32), axis=-1)
    s = jnp.concatenate([sr, sh], axis=-1)

    h = jnp.dot(s, w1_ref[...],
                preferred_element_type=jnp.float32)
    h = jnp.maximum(h * inv_hw + b1_ref[...][None, :], 0.0)

    w2d = w2_ref[0:1, :] - w2_ref[1:2, :]
    b2d = b2_ref[...][0:1] - b2_ref[...][1:2]
    d = jnp.sum(h * w2d, axis=-1, keepdims=True) + b2d
    w_rgb = jax.nn.sigmoid(d)[:, :, None, None]

    out_ref[...] = (hha + w_rgb * (rgb - hha)).astype(out_ref.dtype)


def kernel(rgb, hha, w1, b1, w2, b2):
    assert rgb.shape == hha.shape and rgb.dtype == hha.dtype
    B, C, H, W = rgb.shape

    w1t = w1.T

    NB = 8 if B % 8 == 0 else (4 if B % 4 == 0 else (2 if B % 2 == 0 else 1))

    def park(shape):
        return pl.BlockSpec(shape, lambda b: (0,) * len(shape))

    body = functools.partial(_fused_body, inv_hw=1.0 / (H * W))
    return pl.pallas_call(
        body,
        out_shape=jax.ShapeDtypeStruct((B, C, H, W), rgb.dtype),
        grid=(B // NB,),
        in_specs=[
            pl.BlockSpec((NB, C, H, W), lambda b: (b, 0, 0, 0)),
            pl.BlockSpec((NB, C, H, W), lambda b: (b, 0, 0, 0)),
            park(w1t.shape), park(b1.shape), park(w2.shape), park(b2.shape),
        ],
        out_specs=pl.BlockSpec((NB, C, H, W), lambda b: (b, 0, 0, 0)),
        compiler_params=pltpu.CompilerParams(
            dimension_semantics=("parallel",),
            vmem_limit_bytes=64 * 1024 * 1024),
        cost_estimate=pl.CostEstimate(
            flops=5 * B * C * H * W,
            transcendentals=B,
            bytes_accessed=3 * B * C * H * W * jnp.dtype(rgb.dtype).itemsize),
    )(rgb, hha, w1t, b1, w2, b2)
